# Initial kernel scaffold; baseline (speedup 1.0000x reference)
import jax
import jax.numpy as jnp
from jax.experimental import pallas as pl


def kernel(x_num, x_cat, ln_w, ln_b, proj_w, proj_b, cat_tables, feat_id):
    raise NotImplementedError("write your pallas kernel here")


# speedup vs baseline: 4.2357x; 4.2357x over previous
import functools

import jax
import jax.numpy as jnp
from jax import lax
from jax.experimental import pallas as pl
from jax.experimental.pallas import tpu as pltpu
from jax.experimental.pallas import tpu_sc as plsc

_B = 4096
_NN = 13
_NC = 26
_NF = _NN + _NC
_D = 128
_CARDP = 1001
_STRIDE = 1008
_TROWS = (_NC + 1) * _STRIDE
_RTOT = _B * _NF
_SPW = 104
_NSTREAM = _RTOT // _SPW
_NWORK = 32
_KPW = _NSTREAM // _NWORK
_NBUF = 4


def _bake_body(lnb_ref, pw_ref, pb_ref, fnum_ref, cat_ref, fcat_ref, out_ref):
    i = pl.program_id(0)

    @pl.when(i == 0)
    def _():
        out_ref[0:16, :] = (lnb_ref[...] * pw_ref[...] + pb_ref[...]
                            + fnum_ref[...])
        out_ref[16:, :] = jnp.zeros((_STRIDE - 16, _D), jnp.float32)

    @pl.when(i > 0)
    def _():
        out_ref[0:_CARDP, :] = cat_ref[0] + fcat_ref[0]
        out_ref[_CARDP:, :] = jnp.zeros((_STRIDE - _CARDP, _D), jnp.float32)


def _bake(lnb_b, pw, pb, fnum, cat_tables, fcat):
    return pl.pallas_call(
        _bake_body,
        grid=(_NC + 1,),
        in_specs=[
            pl.BlockSpec((16, _D), lambda i: (0, 0)),
            pl.BlockSpec((16, _D), lambda i: (0, 0)),
            pl.BlockSpec((16, _D), lambda i: (0, 0)),
            pl.BlockSpec((16, _D), lambda i: (0, 0)),
            pl.BlockSpec((1, _CARDP, _D),
                         lambda i: (jnp.maximum(i - 1, 0), 0, 0)),
            pl.BlockSpec((1, 1, _D),
                         lambda i: (jnp.minimum(12 + i, _NF - 1), 0, 0)),
        ],
        out_specs=pl.BlockSpec((_STRIDE, _D), lambda i: (i, 0)),
        out_shape=jax.ShapeDtypeStruct((_TROWS, _D), jnp.float32),
    )(lnb_b, pw, pb, fnum, cat_tables, fcat)


def _idx_body(xs_ref, out_ref):
    col = lax.broadcasted_iota(jnp.int32, (_B, _NF), 1)
    out_ref[...] = jnp.where(col < _NN, col,
                             (col - (_NN - 1)) * _STRIDE + xs_ref[...])


def _make_idx(xshift):
    return pl.pallas_call(
        _idx_body,
        out_shape=jax.ShapeDtypeStruct((_B, _NF), jnp.int32),
    )(xshift)


def _sc_body(idx_hbm, table_hbm, out_hbm, idx_v, b0, b1, b2, b3, gsem, wsem):
    wid = lax.axis_index("s") * 2 + lax.axis_index("c")
    k0 = wid * _KPW
    bufs = [b0, b1, b2, b3]
    pltpu.sync_copy(idx_hbm.at[pl.ds(k0, _KPW)], idx_v)
    for s in range(_NBUF):
        pltpu.async_copy(table_hbm.at[idx_v.at[s]], bufs[s], gsem.at[s])

    def outer(t, carry):
        handles = []
        for s in range(_NBUF):
            k = t * _NBUF + s
            pltpu.make_async_copy(table_hbm.at[idx_v.at[k]], bufs[s],
                                  gsem.at[s]).wait()
            handles.append(pltpu.async_copy(
                bufs[s], out_hbm.at[pl.ds((k0 + k) * _SPW, _SPW)],
                wsem.at[s]))
        for s in range(_NBUF):
            handles[s].wait()
            kn = (t + 1) * _NBUF + s

            @pl.when(kn < _KPW)
            def _(s=s, kn=kn):
                pltpu.async_copy(table_hbm.at[idx_v.at[kn]], bufs[s],
                                 gsem.at[s])
        return carry

    lax.fori_loop(0, _KPW // _NBUF, outer, 0)


def _sc_gather(idx2, flat_table):
    mesh = plsc.VectorSubcoreMesh(core_axis_name="c", subcore_axis_name="s")
    fn = functools.partial(
        pl.kernel,
        mesh=mesh,
        out_type=jax.ShapeDtypeStruct((_RTOT, _D), jnp.float32),
        scratch_types=[
            pltpu.VMEM((_KPW, _SPW), jnp.int32),
            pltpu.VMEM((_SPW, _D), jnp.float32),
            pltpu.VMEM((_SPW, _D), jnp.float32),
            pltpu.VMEM((_SPW, _D), jnp.float32),
            pltpu.VMEM((_SPW, _D), jnp.float32),
            pltpu.SemaphoreType.DMA((_NBUF,)),
            pltpu.SemaphoreType.DMA((_NBUF,)),
        ],
    )(_sc_body)
    return fn(idx2, flat_table)


def kernel(x_num, x_cat, ln_w, ln_b, proj_w, proj_b, cat_tables, feat_id):
    del x_num, ln_w
    f32 = jnp.float32
    lnb_b = jnp.broadcast_to(jnp.pad(ln_b.astype(f32), (0, 3))[:, None],
                             (16, _D))
    pw = jnp.pad(proj_w.astype(f32), ((0, 3), (0, 0)))
    pb = jnp.pad(proj_b.astype(f32), ((0, 3), (0, 0)))
    fnum = jnp.pad(feat_id[:_NN].astype(f32), ((0, 3), (0, 0)))
    fcat = feat_id.astype(f32).reshape(_NF, 1, _D)

    flat_table = _bake(lnb_b, pw, pb, fnum, cat_tables.astype(f32), fcat)

    xshift = jnp.pad(x_cat.astype(jnp.int32), ((0, 0), (_NN, 0)))
    idx = _make_idx(xshift)
    idx2 = idx.reshape(_NSTREAM, _SPW)

    out_flat = _sc_gather(idx2, flat_table)
    return out_flat.reshape(_B, _NF, _D)


# speedup vs baseline: 4.2769x; 1.0097x over previous
import functools

import jax
import jax.numpy as jnp
from jax import lax
from jax.experimental import pallas as pl
from jax.experimental.pallas import tpu as pltpu
from jax.experimental.pallas import tpu_sc as plsc

_B = 4096
_NN = 13
_NC = 26
_NF = _NN + _NC
_D = 128
_CARDP = 1001
_STRIDE = 1008
_TROWS = (_NC + 1) * _STRIDE
_RTOT = _B * _NF
_SPW = 104
_NSTREAM = _RTOT // _SPW
_NWORK = 32
_KPW = _NSTREAM // _NWORK
_BPW = _B // _NWORK
_NBUF = 6


def _bake_body(lnb_ref, pw_ref, pb_ref, fnum_ref, cat_ref, fcat_ref, out_ref):
    i = pl.program_id(0)

    @pl.when(i == 0)
    def _():
        out_ref[0:16, :] = (lnb_ref[...] * pw_ref[...] + pb_ref[...]
                            + fnum_ref[...])
        out_ref[16:, :] = jnp.zeros((_STRIDE - 16, _D), jnp.float32)

    @pl.when(i > 0)
    def _():
        out_ref[0:_CARDP, :] = cat_ref[0] + fcat_ref[0]
        out_ref[_CARDP:, :] = jnp.zeros((_STRIDE - _CARDP, _D), jnp.float32)


def _bake(lnb_b, pw, pb, fnum, cat_tables, fcat):
    return pl.pallas_call(
        _bake_body,
        grid=(_NC + 1,),
        in_specs=[
            pl.BlockSpec((16, _D), lambda i: (0, 0)),
            pl.BlockSpec((16, _D), lambda i: (0, 0)),
            pl.BlockSpec((16, _D), lambda i: (0, 0)),
            pl.BlockSpec((16, _D), lambda i: (0, 0)),
            pl.BlockSpec((1, _CARDP, _D),
                         lambda i: (jnp.maximum(i - 1, 0), 0, 0)),
            pl.BlockSpec((1, 1, _D),
                         lambda i: (jnp.minimum(12 + i, _NF - 1), 0, 0)),
        ],
        out_specs=pl.BlockSpec((_STRIDE, _D), lambda i: (i, 0)),
        out_shape=jax.ShapeDtypeStruct((_TROWS, _D), jnp.float32),
    )(lnb_b, pw, pb, fnum, cat_tables, fcat)


def _sc_body(xcat_hbm, table_hbm, out_hbm, xc_v, idx_v, *rest):
    bufs = list(rest[:_NBUF])
    gsem, wsem = rest[_NBUF], rest[_NBUF + 1]
    wid = lax.axis_index("s") * 2 + lax.axis_index("c")
    k0 = wid * _KPW
    b0 = wid * _BPW
    pltpu.sync_copy(xcat_hbm.at[pl.ds(b0 * 128, _BPW * 128)], xc_v)

    lane = lax.iota(jnp.int32, 16)
    sv0 = (lane + 1) * _STRIDE
    sv1 = (lane + 17) * _STRIDE
    def row_fn(r, carry):
        base = r * _NF
        xr0 = xc_v[pl.ds(r * 128, 16)]
        xr1 = xc_v[pl.ds(r * 128 + 16, 16)]
        idx_v[pl.ds(base, 16)] = lane
        idx_v[pl.ds(base + _NN, 16)] = sv0 + xr0
        idx_v[pl.ds(base + _NN + 16, 16)] = sv1 + xr1
        return carry
    lax.fori_loop(0, _BPW, row_fn, 0)

    for s in range(_NBUF):
        pltpu.async_copy(table_hbm.at[idx_v.at[pl.ds(s * _SPW, _SPW)]],
                         bufs[s], gsem.at[s])

    def outer(t, carry):
        handles = []
        for s in range(_NBUF):
            k = t * _NBUF + s
            pltpu.make_async_copy(
                table_hbm.at[idx_v.at[pl.ds(k * _SPW, _SPW)]], bufs[s],
                gsem.at[s]).wait()
            handles.append(pltpu.async_copy(
                bufs[s], out_hbm.at[pl.ds((k0 + k) * _SPW, _SPW)],
                wsem.at[s]))
        for s in range(_NBUF):
            handles[s].wait()
            kn = (t + 1) * _NBUF + s

            @pl.when(kn < _KPW)
            def _(s=s, kn=kn):
                pltpu.async_copy(
                    table_hbm.at[idx_v.at[pl.ds(kn * _SPW, _SPW)]], bufs[s],
                    gsem.at[s])
        return carry

    lax.fori_loop(0, _KPW // _NBUF, outer, 0)


def _sc_gather(xcat_pad, flat_table):
    mesh = plsc.VectorSubcoreMesh(core_axis_name="c", subcore_axis_name="s")
    fn = functools.partial(
        pl.kernel,
        mesh=mesh,
        out_type=jax.ShapeDtypeStruct((_RTOT, _D), jnp.float32),
        scratch_types=(
            [pltpu.VMEM((_BPW * 128,), jnp.int32),
             pltpu.VMEM((_BPW * _NF + 16,), jnp.int32)]
            + [pltpu.VMEM((_SPW, _D), jnp.float32) for _ in range(_NBUF)]
            + [pltpu.SemaphoreType.DMA((_NBUF,)),
               pltpu.SemaphoreType.DMA((_NBUF,))]
        ),
    )(_sc_body)
    return fn(xcat_pad, flat_table)


def kernel(x_num, x_cat, ln_w, ln_b, proj_w, proj_b, cat_tables, feat_id):
    del x_num, ln_w
    f32 = jnp.float32
    lnb_b = jnp.broadcast_to(jnp.pad(ln_b.astype(f32), (0, 3))[:, None],
                             (16, _D))
    pw = jnp.pad(proj_w.astype(f32), ((0, 3), (0, 0)))
    pb = jnp.pad(proj_b.astype(f32), ((0, 3), (0, 0)))
    fnum = jnp.pad(feat_id[:_NN].astype(f32), ((0, 3), (0, 0)))
    fcat = feat_id.astype(f32).reshape(_NF, 1, _D)

    flat_table = _bake(lnb_b, pw, pb, fnum, cat_tables.astype(f32), fcat)

    xcat_pad = jnp.pad(x_cat.astype(jnp.int32),
                       ((0, 0), (0, 128 - _NC))).reshape(_B * 128)
    out_flat = _sc_gather(xcat_pad, flat_table)
    return out_flat.reshape(_B, _NF, _D)
